# baseline (speedup 1.0000x reference)
import jax
import jax.numpy as jnp
from jax import lax
from jax.experimental import pallas as pl

M_ITEM = 1_000_000
DIM = 64
MB = 4_096
ROWS = MB * 8


def _dot_block(items_ref, user_ref, out_ref):
    u = user_ref[...]
    x = items_ref[...]
    x3 = x.reshape(ROWS // 128, 128, DIM)
    y = lax.dot_general(
        u, x3, (((1,), (2,)), ((), ())), preferred_element_type=jnp.float32
    )
    out_ref[...] = y.reshape(ROWS)


def kernel(items_emb, user_emb):
    n = items_emb.shape[0]
    x3 = items_emb.reshape(n // 8, 8, DIM)
    grid = (n // 8 + MB - 1) // MB
    return pl.pallas_call(
        _dot_block,
        grid=(grid,),
        in_specs=[
            pl.BlockSpec((MB, 8, DIM), lambda i: (i, 0, 0)),
            pl.BlockSpec((1, DIM), lambda i: (0, 0)),
        ],
        out_specs=pl.BlockSpec((ROWS,), lambda i: (i,)),
        out_shape=jax.ShapeDtypeStruct((n,), items_emb.dtype),
    )(x3, user_emb)


# speedup vs baseline: 1.0026x; 1.0026x over previous
import jax
import jax.numpy as jnp
from jax import lax
from jax.experimental import pallas as pl

M_ITEM = 1_000_000
DIM = 64
MB = 2_048
ROWS = MB * 8


def _dot_block(items_ref, user_ref, out_ref):
    u = user_ref[...]
    x = items_ref[...]
    x3 = x.reshape(ROWS // 128, 128, DIM)
    y = lax.dot_general(
        u, x3, (((1,), (2,)), ((), ())), preferred_element_type=jnp.float32
    )
    out_ref[...] = y.reshape(ROWS)


def kernel(items_emb, user_emb):
    n = items_emb.shape[0]
    x3 = items_emb.reshape(n // 8, 8, DIM)
    grid = (n // 8 + MB - 1) // MB
    return pl.pallas_call(
        _dot_block,
        grid=(grid,),
        in_specs=[
            pl.BlockSpec((MB, 8, DIM), lambda i: (i, 0, 0)),
            pl.BlockSpec((1, DIM), lambda i: (0, 0)),
        ],
        out_specs=pl.BlockSpec((ROWS,), lambda i: (i,)),
        out_shape=jax.ShapeDtypeStruct((n,), items_emb.dtype),
    )(x3, user_emb)
